# Initial kernel scaffold; baseline (speedup 1.0000x reference)
import functools

import jax
import jax.numpy as jnp
from jax import lax
from jax.experimental import pallas as pl


B, N, S, D1, D2 = 8, 4096, 1024, 256, 256
H0, H1 = 512, 256
TN = 256
NT = N // TN
CNT = float(B * N)
EPS = 1e-5
BIG = jnp.float32(3.0e38)


def _k1_body(xyz1_ref, xyz2_ref, p1_ref, p2_ref, w0_ref,
             z0_ref, s0_ref, q0_ref):
    b = pl.program_id(0)
    nt = pl.program_id(1)

    x1 = xyz1_ref[0]
    x2 = xyz2_ref[0]

    x2sq = jnp.sum(x2 * x2, axis=0, keepdims=True)
    x1sq = jnp.sum(x1 * x1, axis=0, keepdims=True)
    ones_s = jnp.ones((1, S), jnp.float32)
    ones_n = jnp.ones((1, TN), jnp.float32)
    A = jnp.concatenate([x2, x2sq, ones_s], axis=0)
    Bm = jnp.concatenate([-2.0 * x1, ones_n, x1sq], axis=0)
    dT = lax.dot_general(A, Bm, (((0,), (0,)), ((), ())),
                         preferred_element_type=jnp.float32)

    iota = lax.broadcasted_iota(jnp.int32, (S, TN), 0)

    m1 = jnp.min(dT, axis=0, keepdims=True)
    i1 = jnp.min(jnp.where(dT == m1, iota, S), axis=0, keepdims=True)
    d2 = jnp.where(iota == i1, BIG, dT)
    m2 = jnp.min(d2, axis=0, keepdims=True)
    i2 = jnp.min(jnp.where(d2 == m2, iota, S), axis=0, keepdims=True)
    d3 = jnp.where(iota == i2, BIG, d2)
    m3 = jnp.min(d3, axis=0, keepdims=True)
    i3 = jnp.min(jnp.where(d3 == m3, iota, S), axis=0, keepdims=True)

    r1 = 1.0 / jnp.maximum(m1, 1e-8)
    r2 = 1.0 / jnp.maximum(m2, 1e-8)
    r3 = 1.0 / jnp.maximum(m3, 1e-8)
    norm = jnp.maximum(r1 + r2 + r3, 1e-8)
    w1 = r1 / norm
    w2 = r2 / norm
    w3 = r3 / norm

    zero = jnp.zeros((), jnp.float32)
    Wsel = (jnp.where(iota == i1, w1, zero)
            + jnp.where(iota == i2, w2, zero)
            + jnp.where(iota == i3, w3, zero))

    interp = jnp.dot(p2_ref[0], Wsel,
                     preferred_element_type=jnp.float32)

    npts = jnp.concatenate([p1_ref[0], interp], axis=0)
    z0 = jnp.dot(w0_ref[...], npts,
                 preferred_element_type=jnp.float32)
    z0_ref[0] = z0

    @pl.when(jnp.logical_and(b == 0, nt == 0))
    def _():
        s0_ref[...] = jnp.zeros_like(s0_ref)
        q0_ref[...] = jnp.zeros_like(q0_ref)

    s0_ref[...] += jnp.sum(z0, axis=1, keepdims=True)
    q0_ref[...] += jnp.sum(z0 * z0, axis=1, keepdims=True)


def _k2_body(z0_ref, s0_ref, q0_ref, g0_ref, b0_ref, w1_ref,
             z1_ref, s1_ref, q1_ref):
    b = pl.program_id(0)
    nt = pl.program_id(1)

    mean = s0_ref[...] * (1.0 / CNT)
    var = q0_ref[...] * (1.0 / CNT) - mean * mean
    a = g0_ref[...] * lax.rsqrt(var + EPS)
    c = b0_ref[...] - mean * a
    h = jnp.maximum(z0_ref[0] * a + c, 0.0)
    z1 = jnp.dot(w1_ref[...], h,
                 preferred_element_type=jnp.float32)
    z1_ref[0] = z1

    @pl.when(jnp.logical_and(b == 0, nt == 0))
    def _():
        s1_ref[...] = jnp.zeros_like(s1_ref)
        q1_ref[...] = jnp.zeros_like(q1_ref)

    s1_ref[...] += jnp.sum(z1, axis=1, keepdims=True)
    q1_ref[...] += jnp.sum(z1 * z1, axis=1, keepdims=True)


def _k3_body(z1_ref, s1_ref, q1_ref, g1_ref, b1_ref, out_ref):
    mean = s1_ref[...] * (1.0 / CNT)
    var = q1_ref[...] * (1.0 / CNT) - mean * mean
    a = g1_ref[...] * lax.rsqrt(var + EPS)
    c = b1_ref[...] - mean * a
    out_ref[0] = jnp.maximum(z1_ref[0] * a + c, 0.0)


@jax.jit
def kernel(xyz1, xyz2, points1, points2, W0, g0, b0, W1, g1, b1):
    g0c = g0.reshape(H0, 1)
    b0c = b0.reshape(H0, 1)
    g1c = g1.reshape(H1, 1)
    b1c = b1.reshape(H1, 1)

    z0, s0, q0 = pl.pallas_call(
        _k1_body,
        grid=(B, NT),
        in_specs=[
            pl.BlockSpec((1, 3, TN), lambda b, n: (b, 0, n)),
            pl.BlockSpec((1, 3, S), lambda b, n: (b, 0, 0)),
            pl.BlockSpec((1, D1, TN), lambda b, n: (b, 0, n)),
            pl.BlockSpec((1, D2, S), lambda b, n: (b, 0, 0)),
            pl.BlockSpec((H0, D1 + D2), lambda b, n: (0, 0)),
        ],
        out_specs=[
            pl.BlockSpec((1, H0, TN), lambda b, n: (b, 0, n)),
            pl.BlockSpec((H0, 1), lambda b, n: (0, 0)),
            pl.BlockSpec((H0, 1), lambda b, n: (0, 0)),
        ],
        out_shape=[
            jax.ShapeDtypeStruct((B, H0, N), jnp.float32),
            jax.ShapeDtypeStruct((H0, 1), jnp.float32),
            jax.ShapeDtypeStruct((H0, 1), jnp.float32),
        ],
    )(xyz1, xyz2, points1, points2, W0)

    z1, s1, q1 = pl.pallas_call(
        _k2_body,
        grid=(B, NT),
        in_specs=[
            pl.BlockSpec((1, H0, TN), lambda b, n: (b, 0, n)),
            pl.BlockSpec((H0, 1), lambda b, n: (0, 0)),
            pl.BlockSpec((H0, 1), lambda b, n: (0, 0)),
            pl.BlockSpec((H0, 1), lambda b, n: (0, 0)),
            pl.BlockSpec((H0, 1), lambda b, n: (0, 0)),
            pl.BlockSpec((H1, H0), lambda b, n: (0, 0)),
        ],
        out_specs=[
            pl.BlockSpec((1, H1, TN), lambda b, n: (b, 0, n)),
            pl.BlockSpec((H1, 1), lambda b, n: (0, 0)),
            pl.BlockSpec((H1, 1), lambda b, n: (0, 0)),
        ],
        out_shape=[
            jax.ShapeDtypeStruct((B, H1, N), jnp.float32),
            jax.ShapeDtypeStruct((H1, 1), jnp.float32),
            jax.ShapeDtypeStruct((H1, 1), jnp.float32),
        ],
    )(z0, s0, q0, g0c, b0c, W1)

    out = pl.pallas_call(
        _k3_body,
        grid=(B, NT),
        in_specs=[
            pl.BlockSpec((1, H1, TN), lambda b, n: (b, 0, n)),
            pl.BlockSpec((H1, 1), lambda b, n: (0, 0)),
            pl.BlockSpec((H1, 1), lambda b, n: (0, 0)),
            pl.BlockSpec((H1, 1), lambda b, n: (0, 0)),
            pl.BlockSpec((H1, 1), lambda b, n: (0, 0)),
        ],
        out_specs=pl.BlockSpec((1, H1, TN), lambda b, n: (b, 0, n)),
        out_shape=jax.ShapeDtypeStruct((B, H1, N), jnp.float32),
    )(z1, s1, q1, g1c, b1c)

    return out


# speedup vs baseline: 7.6324x; 7.6324x over previous
import functools

import jax
import jax.numpy as jnp
from jax import lax
from jax.experimental import pallas as pl


B, N, S, D1, D2 = 8, 4096, 1024, 256, 256
H0, H1 = 512, 256
TN = 256
NT = N // TN
CNT = float(B * N)
EPS = 1e-5
BIG = 3.0e38


def _k1_body(xyz1_ref, xyz2_ref, p1_ref, p2_ref, w0_ref,
             z0_ref, s0_ref, q0_ref):
    b = pl.program_id(0)
    nt = pl.program_id(1)

    x1 = xyz1_ref[0]
    x2 = xyz2_ref[0]

    x1r = x1.astype(jnp.bfloat16).astype(jnp.float32)
    x2r = x2.astype(jnp.bfloat16).astype(jnp.float32)
    cross = lax.dot_general(x2r, x1r, (((0,), (0,)), ((), ())),
                            preferred_element_type=jnp.float32,
                            precision=lax.Precision.HIGHEST)
    x1sq = jnp.sum(x1 * x1, axis=0, keepdims=True)
    x2sq_col = lax.dot_general(
        x2 * x2, jnp.ones((3, 1), jnp.float32), (((0,), (0,)), ((), ())),
        preferred_element_type=jnp.float32,
        precision=lax.Precision.HIGHEST)
    dT = (x1sq + x2sq_col) - 2.0 * cross

    iota = lax.broadcasted_iota(jnp.int32, (S, TN), 0)

    m1 = jnp.min(dT, axis=0, keepdims=True)
    i1 = jnp.min(jnp.where(dT == m1, iota, S), axis=0, keepdims=True)
    d2 = jnp.where(iota == i1, BIG, dT)
    m2 = jnp.min(d2, axis=0, keepdims=True)
    i2 = jnp.min(jnp.where(d2 == m2, iota, S), axis=0, keepdims=True)
    d3 = jnp.where(iota == i2, BIG, d2)
    m3 = jnp.min(d3, axis=0, keepdims=True)
    i3 = jnp.min(jnp.where(d3 == m3, iota, S), axis=0, keepdims=True)

    r1 = 1.0 / jnp.maximum(m1, 1e-8)
    r2 = 1.0 / jnp.maximum(m2, 1e-8)
    r3 = 1.0 / jnp.maximum(m3, 1e-8)
    norm = jnp.maximum(r1 + r2 + r3, 1e-8)
    w1 = r1 / norm
    w2 = r2 / norm
    w3 = r3 / norm

    zero = jnp.zeros((), jnp.float32)
    Wsel = (jnp.where(iota == i1, w1, zero)
            + jnp.where(iota == i2, w2, zero)
            + jnp.where(iota == i3, w3, zero))

    interp = jnp.dot(p2_ref[0], Wsel,
                     preferred_element_type=jnp.float32,
                     precision=lax.Precision.HIGHEST)

    npts = jnp.concatenate([p1_ref[0], interp], axis=0)
    z0 = jnp.dot(w0_ref[...], npts,
                 preferred_element_type=jnp.float32,
                 precision=lax.Precision.HIGHEST)
    z0_ref[0] = z0

    @pl.when(jnp.logical_and(b == 0, nt == 0))
    def _():
        s0_ref[...] = jnp.zeros_like(s0_ref)
        q0_ref[...] = jnp.zeros_like(q0_ref)

    s0_ref[...] += jnp.sum(z0, axis=1, keepdims=True)
    q0_ref[...] += jnp.sum(z0 * z0, axis=1, keepdims=True)


def _k2_body(z0_ref, s0_ref, q0_ref, g0_ref, b0_ref, w1_ref,
             z1_ref, s1_ref, q1_ref):
    b = pl.program_id(0)
    nt = pl.program_id(1)

    mean = s0_ref[...] * (1.0 / CNT)
    var = q0_ref[...] * (1.0 / CNT) - mean * mean
    a = g0_ref[...] * lax.rsqrt(var + EPS)
    c = b0_ref[...] - mean * a
    h = jnp.maximum(z0_ref[0] * a + c, 0.0)
    z1 = jnp.dot(w1_ref[...], h,
                 preferred_element_type=jnp.float32,
                 precision=lax.Precision.HIGHEST)
    z1_ref[0] = z1

    @pl.when(jnp.logical_and(b == 0, nt == 0))
    def _():
        s1_ref[...] = jnp.zeros_like(s1_ref)
        q1_ref[...] = jnp.zeros_like(q1_ref)

    s1_ref[...] += jnp.sum(z1, axis=1, keepdims=True)
    q1_ref[...] += jnp.sum(z1 * z1, axis=1, keepdims=True)


def _k3_body(z1_ref, s1_ref, q1_ref, g1_ref, b1_ref, out_ref):
    mean = s1_ref[...] * (1.0 / CNT)
    var = q1_ref[...] * (1.0 / CNT) - mean * mean
    a = g1_ref[...] * lax.rsqrt(var + EPS)
    c = b1_ref[...] - mean * a
    out_ref[0] = jnp.maximum(z1_ref[0] * a + c, 0.0)


@jax.jit
def kernel(xyz1, xyz2, points1, points2, W0, g0, b0, W1, g1, b1):
    g0c = g0.reshape(H0, 1)
    b0c = b0.reshape(H0, 1)
    g1c = g1.reshape(H1, 1)
    b1c = b1.reshape(H1, 1)

    z0, s0, q0 = pl.pallas_call(
        _k1_body,
        grid=(B, NT),
        in_specs=[
            pl.BlockSpec((1, 3, TN), lambda b, n: (b, 0, n)),
            pl.BlockSpec((1, 3, S), lambda b, n: (b, 0, 0)),
            pl.BlockSpec((1, D1, TN), lambda b, n: (b, 0, n)),
            pl.BlockSpec((1, D2, S), lambda b, n: (b, 0, 0)),
            pl.BlockSpec((H0, D1 + D2), lambda b, n: (0, 0)),
        ],
        out_specs=[
            pl.BlockSpec((1, H0, TN), lambda b, n: (b, 0, n)),
            pl.BlockSpec((H0, 1), lambda b, n: (0, 0)),
            pl.BlockSpec((H0, 1), lambda b, n: (0, 0)),
        ],
        out_shape=[
            jax.ShapeDtypeStruct((B, H0, N), jnp.float32),
            jax.ShapeDtypeStruct((H0, 1), jnp.float32),
            jax.ShapeDtypeStruct((H0, 1), jnp.float32),
        ],
    )(xyz1, xyz2, points1, points2, W0)

    z1, s1, q1 = pl.pallas_call(
        _k2_body,
        grid=(B, NT),
        in_specs=[
            pl.BlockSpec((1, H0, TN), lambda b, n: (b, 0, n)),
            pl.BlockSpec((H0, 1), lambda b, n: (0, 0)),
            pl.BlockSpec((H0, 1), lambda b, n: (0, 0)),
            pl.BlockSpec((H0, 1), lambda b, n: (0, 0)),
            pl.BlockSpec((H0, 1), lambda b, n: (0, 0)),
            pl.BlockSpec((H1, H0), lambda b, n: (0, 0)),
        ],
        out_specs=[
            pl.BlockSpec((1, H1, TN), lambda b, n: (b, 0, n)),
            pl.BlockSpec((H1, 1), lambda b, n: (0, 0)),
            pl.BlockSpec((H1, 1), lambda b, n: (0, 0)),
        ],
        out_shape=[
            jax.ShapeDtypeStruct((B, H1, N), jnp.float32),
            jax.ShapeDtypeStruct((H1, 1), jnp.float32),
            jax.ShapeDtypeStruct((H1, 1), jnp.float32),
        ],
    )(z0, s0, q0, g0c, b0c, W1)

    out = pl.pallas_call(
        _k3_body,
        grid=(B, NT),
        in_specs=[
            pl.BlockSpec((1, H1, TN), lambda b, n: (b, 0, n)),
            pl.BlockSpec((H1, 1), lambda b, n: (0, 0)),
            pl.BlockSpec((H1, 1), lambda b, n: (0, 0)),
            pl.BlockSpec((H1, 1), lambda b, n: (0, 0)),
            pl.BlockSpec((H1, 1), lambda b, n: (0, 0)),
        ],
        out_specs=pl.BlockSpec((1, H1, TN), lambda b, n: (b, 0, n)),
        out_shape=jax.ShapeDtypeStruct((B, H1, N), jnp.float32),
    )(z1, s1, q1, g1c, b1c)

    return out


# speedup vs baseline: 8.6919x; 1.1388x over previous
import functools

import jax
import jax.numpy as jnp
from jax import lax
from jax.experimental import pallas as pl
from jax.experimental.pallas import tpu as pltpu, tpu_sc as plsc


B, N, S, D1, D2 = 8, 4096, 1024, 256, 256
H0, H1 = 512, 256
TN = 256
NT = N // TN
CNT = float(B * N)
EPS = 1e-5
BIG = 3.0e38

NC, NS, L = 2, 16, 16
NW = NC * NS
QPW = (B * N) // NW
Q = 64
NCHUNK = QPW // Q


def _k1_body(xyz1_ref, xyz2_ref, p2_ref,
             wa_ref, wb_ref, wc_ref, ga_ref, gb_ref, gc_ref, p2t_ref):
    b = pl.program_id(0)
    nt = pl.program_id(1)

    x1 = xyz1_ref[0]
    x2 = xyz2_ref[0]

    x1r = x1.astype(jnp.bfloat16).astype(jnp.float32)
    x2r = x2.astype(jnp.bfloat16).astype(jnp.float32)
    x1sq = jnp.sum(x1 * x1, axis=0, keepdims=True)
    cross = lax.dot_general(x2r, x1r, (((0,), (0,)), ((), ())),
                            preferred_element_type=jnp.float32,
                            precision=lax.Precision.HIGHEST)
    x2sq_col = lax.dot_general(
        x2 * x2, jnp.ones((3, 1), jnp.float32), (((0,), (0,)), ((), ())),
        preferred_element_type=jnp.float32,
        precision=lax.Precision.HIGHEST)
    dT = (x1sq + x2sq_col) - 2.0 * cross

    iota = lax.broadcasted_iota(jnp.int32, (S, TN), 0)

    m1 = jnp.min(dT, axis=0, keepdims=True)
    i1 = jnp.min(jnp.where(dT == m1, iota, S), axis=0, keepdims=True)
    d2 = jnp.where(iota == i1, BIG, dT)
    m2 = jnp.min(d2, axis=0, keepdims=True)
    i2 = jnp.min(jnp.where(d2 == m2, iota, S), axis=0, keepdims=True)
    d3 = jnp.where(iota == i2, BIG, d2)
    m3 = jnp.min(d3, axis=0, keepdims=True)
    i3 = jnp.min(jnp.where(d3 == m3, iota, S), axis=0, keepdims=True)

    r1 = 1.0 / jnp.maximum(m1, 1e-8)
    r2 = 1.0 / jnp.maximum(m2, 1e-8)
    r3 = 1.0 / jnp.maximum(m3, 1e-8)
    norm = jnp.maximum(r1 + r2 + r3, 1e-8)

    wa_ref[0] = r1 / norm
    wb_ref[0] = r2 / norm
    wc_ref[0] = r3 / norm
    off = b * S
    ga_ref[0] = i1 + off
    gb_ref[0] = i2 + off
    gc_ref[0] = i3 + off

    @pl.when(nt == 0)
    def _():
        p2t_ref[...] = jnp.transpose(p2_ref[0], (1, 0))


def _sc_body(wa_hbm, wb_hbm, wc_hbm, ga_hbm, gb_hbm, gc_hbm, tbl_hbm,
             out_hbm, i0_v, i1_v, i2_v, w0_v, w1_v, w2_v, rows_v, out_v, sem):
    wid = lax.axis_index("s") * NC + lax.axis_index("c")
    qbase0 = wid * QPW

    pltpu.sync_copy(ga_hbm.at[pl.ds(qbase0, QPW)], i0_v)
    pltpu.sync_copy(gb_hbm.at[pl.ds(qbase0, QPW)], i1_v)
    pltpu.sync_copy(gc_hbm.at[pl.ds(qbase0, QPW)], i2_v)
    pltpu.sync_copy(wa_hbm.at[pl.ds(qbase0, QPW)], w0_v.at[pl.ds(0, QPW)])
    pltpu.sync_copy(wb_hbm.at[pl.ds(qbase0, QPW)], w1_v.at[pl.ds(0, QPW)])
    pltpu.sync_copy(wc_hbm.at[pl.ds(qbase0, QPW)], w2_v.at[pl.ds(0, QPW)])

    def chunk(c, carry):
        qbase = qbase0 + c * Q
        q0 = c * Q
        cp0 = pltpu.async_copy(tbl_hbm.at[i0_v.at[pl.ds(q0, Q)]],
                               rows_v.at[0], sem)
        cp1 = pltpu.async_copy(tbl_hbm.at[i1_v.at[pl.ds(q0, Q)]],
                               rows_v.at[1], sem)
        cp2 = pltpu.async_copy(tbl_hbm.at[i2_v.at[pl.ds(q0, Q)]],
                               rows_v.at[2], sem)
        cp0.wait()
        cp1.wait()
        cp2.wait()

        def per_q(q, carry2):
            w0 = w0_v[pl.ds(q0 + q, L)][0]
            w1 = w1_v[pl.ds(q0 + q, L)][0]
            w2 = w2_v[pl.ds(q0 + q, L)][0]
            for dd in range(D2 // L):
                sl = pl.ds(dd * L, L)
                acc = (w0 * rows_v[0, q, sl] + w1 * rows_v[1, q, sl]
                       + w2 * rows_v[2, q, sl])
                out_v[q, sl] = acc
            return carry2

        lax.fori_loop(0, Q, per_q, 0)
        pltpu.sync_copy(out_v, out_hbm.at[pl.ds(qbase, Q)])
        return carry

    lax.fori_loop(0, NCHUNK, chunk, 0)


def _k2_body(p1_ref, it_ref, w0_ref, z0_ref, s0_ref, q0_ref):
    b = pl.program_id(0)
    nt = pl.program_id(1)
    w0a = w0_ref[...][:, :D1]
    w0b = w0_ref[...][:, D1:]
    z0 = jnp.dot(w0a, p1_ref[0], preferred_element_type=jnp.float32,
                 precision=lax.Precision.HIGHEST)
    z0 = z0 + lax.dot_general(w0b, it_ref[...], (((1,), (1,)), ((), ())),
                              preferred_element_type=jnp.float32,
                              precision=lax.Precision.HIGHEST)
    z0_ref[0] = z0

    @pl.when(jnp.logical_and(b == 0, nt == 0))
    def _():
        s0_ref[...] = jnp.zeros_like(s0_ref)
        q0_ref[...] = jnp.zeros_like(q0_ref)

    s0_ref[...] += jnp.sum(z0, axis=1, keepdims=True)
    q0_ref[...] += jnp.sum(z0 * z0, axis=1, keepdims=True)


def _k3_body(z0_ref, s0_ref, q0_ref, g0_ref, b0_ref, w1_ref,
             z1_ref, s1_ref, q1_ref):
    b = pl.program_id(0)
    nt = pl.program_id(1)
    mean = s0_ref[...] * (1.0 / CNT)
    var = q0_ref[...] * (1.0 / CNT) - mean * mean
    a = g0_ref[...] * lax.rsqrt(var + EPS)
    c = b0_ref[...] - mean * a
    h = jnp.maximum(z0_ref[0] * a + c, 0.0)
    z1 = jnp.dot(w1_ref[...], h, preferred_element_type=jnp.float32,
                 precision=lax.Precision.HIGHEST)
    z1_ref[0] = z1

    @pl.when(jnp.logical_and(b == 0, nt == 0))
    def _():
        s1_ref[...] = jnp.zeros_like(s1_ref)
        q1_ref[...] = jnp.zeros_like(q1_ref)

    s1_ref[...] += jnp.sum(z1, axis=1, keepdims=True)
    q1_ref[...] += jnp.sum(z1 * z1, axis=1, keepdims=True)


def _k4_body(z1_ref, s1_ref, q1_ref, g1_ref, b1_ref, out_ref):
    mean = s1_ref[...] * (1.0 / CNT)
    var = q1_ref[...] * (1.0 / CNT) - mean * mean
    a = g1_ref[...] * lax.rsqrt(var + EPS)
    c = b1_ref[...] - mean * a
    out_ref[0] = jnp.maximum(z1_ref[0] * a + c, 0.0)


@jax.jit
def kernel(xyz1, xyz2, points1, points2, W0, g0, b0, W1, g1, b1):
    g0c = g0.reshape(H0, 1)
    b0c = b0.reshape(H0, 1)
    g1c = g1.reshape(H1, 1)
    b1c = b1.reshape(H1, 1)

    wa, wb, wc, ga, gb, gc, p2t = pl.pallas_call(
        _k1_body,
        grid=(B, NT),
        in_specs=[
            pl.BlockSpec((1, 3, TN), lambda b, n: (b, 0, n)),
            pl.BlockSpec((1, 3, S), lambda b, n: (b, 0, 0)),
            pl.BlockSpec((1, D2, S), lambda b, n: (b, 0, 0)),
        ],
        out_specs=[
            pl.BlockSpec((1, 1, TN), lambda b, n: (b, 0, n)),
            pl.BlockSpec((1, 1, TN), lambda b, n: (b, 0, n)),
            pl.BlockSpec((1, 1, TN), lambda b, n: (b, 0, n)),
            pl.BlockSpec((1, 1, TN), lambda b, n: (b, 0, n)),
            pl.BlockSpec((1, 1, TN), lambda b, n: (b, 0, n)),
            pl.BlockSpec((1, 1, TN), lambda b, n: (b, 0, n)),
            pl.BlockSpec((S, D2), lambda b, n: (b, 0)),
        ],
        out_shape=[
            jax.ShapeDtypeStruct((B, 1, N), jnp.float32),
            jax.ShapeDtypeStruct((B, 1, N), jnp.float32),
            jax.ShapeDtypeStruct((B, 1, N), jnp.float32),
            jax.ShapeDtypeStruct((B, 1, N), jnp.int32),
            jax.ShapeDtypeStruct((B, 1, N), jnp.int32),
            jax.ShapeDtypeStruct((B, 1, N), jnp.int32),
            jax.ShapeDtypeStruct((B * S, D2), jnp.float32),
        ],
    )(xyz1, xyz2, points2)

    mesh = plsc.VectorSubcoreMesh(core_axis_name="c", subcore_axis_name="s")
    waf = wa.reshape(B * N)
    wbf = wb.reshape(B * N)
    wcf = wc.reshape(B * N)
    gaf = ga.reshape(B * N)
    gbf = gb.reshape(B * N)
    gcf = gc.reshape(B * N)
    interp = pl.kernel(
        _sc_body,
        mesh=mesh,
        out_type=jax.ShapeDtypeStruct((B * N, D2), jnp.float32),
        scratch_types=[
            pltpu.VMEM((QPW,), jnp.int32),
            pltpu.VMEM((QPW,), jnp.int32),
            pltpu.VMEM((QPW,), jnp.int32),
            pltpu.VMEM((QPW + L,), jnp.float32),
            pltpu.VMEM((QPW + L,), jnp.float32),
            pltpu.VMEM((QPW + L,), jnp.float32),
            pltpu.VMEM((3, Q, D2), jnp.float32),
            pltpu.VMEM((Q, D2), jnp.float32),
            pltpu.SemaphoreType.DMA,
        ],
    )(waf, wbf, wcf, gaf, gbf, gcf, p2t)

    z0, s0, q0 = pl.pallas_call(
        _k2_body,
        grid=(B, NT),
        in_specs=[
            pl.BlockSpec((1, D1, TN), lambda b, n: (b, 0, n)),
            pl.BlockSpec((TN, D2), lambda b, n: (b * NT + n, 0)),
            pl.BlockSpec((H0, D1 + D2), lambda b, n: (0, 0)),
        ],
        out_specs=[
            pl.BlockSpec((1, H0, TN), lambda b, n: (b, 0, n)),
            pl.BlockSpec((H0, 1), lambda b, n: (0, 0)),
            pl.BlockSpec((H0, 1), lambda b, n: (0, 0)),
        ],
        out_shape=[
            jax.ShapeDtypeStruct((B, H0, N), jnp.float32),
            jax.ShapeDtypeStruct((H0, 1), jnp.float32),
            jax.ShapeDtypeStruct((H0, 1), jnp.float32),
        ],
    )(points1, interp, W0)

    z1, s1, q1 = pl.pallas_call(
        _k3_body,
        grid=(B, NT),
        in_specs=[
            pl.BlockSpec((1, H0, TN), lambda b, n: (b, 0, n)),
            pl.BlockSpec((H0, 1), lambda b, n: (0, 0)),
            pl.BlockSpec((H0, 1), lambda b, n: (0, 0)),
            pl.BlockSpec((H0, 1), lambda b, n: (0, 0)),
            pl.BlockSpec((H0, 1), lambda b, n: (0, 0)),
            pl.BlockSpec((H1, H0), lambda b, n: (0, 0)),
        ],
        out_specs=[
            pl.BlockSpec((1, H1, TN), lambda b, n: (b, 0, n)),
            pl.BlockSpec((H1, 1), lambda b, n: (0, 0)),
            pl.BlockSpec((H1, 1), lambda b, n: (0, 0)),
        ],
        out_shape=[
            jax.ShapeDtypeStruct((B, H1, N), jnp.float32),
            jax.ShapeDtypeStruct((H1, 1), jnp.float32),
            jax.ShapeDtypeStruct((H1, 1), jnp.float32),
        ],
    )(z0, s0, q0, g0c, b0c, W1)

    out = pl.pallas_call(
        _k4_body,
        grid=(B, NT),
        in_specs=[
            pl.BlockSpec((1, H1, TN), lambda b, n: (b, 0, n)),
            pl.BlockSpec((H1, 1), lambda b, n: (0, 0)),
            pl.BlockSpec((H1, 1), lambda b, n: (0, 0)),
            pl.BlockSpec((H1, 1), lambda b, n: (0, 0)),
            pl.BlockSpec((H1, 1), lambda b, n: (0, 0)),
        ],
        out_specs=pl.BlockSpec((1, H1, TN), lambda b, n: (b, 0, n)),
        out_shape=jax.ShapeDtypeStruct((B, H1, N), jnp.float32),
    )(z1, s1, q1, g1c, b1c)

    return out


# speedup vs baseline: 10.6478x; 1.2250x over previous
import functools

import jax
import jax.numpy as jnp
from jax import lax
from jax.experimental import pallas as pl
from jax.experimental.pallas import tpu as pltpu, tpu_sc as plsc


B, N, S, D1, D2 = 8, 4096, 1024, 256, 256
H0, H1 = 512, 256
TN = 256
NT = N // TN
CNT = float(B * N)
EPS = 1e-5
BIG = 3.0e38

NC, NS, L = 2, 16, 16
NW = NC * NS
QPW = (B * N) // NW
Q = 64
NCHUNK = QPW // Q


def _k1_body(xyz1_ref, xyz2_ref, p2_ref,
             wa_ref, wb_ref, wc_ref, ga_ref, gb_ref, gc_ref, p2t_ref):
    b = pl.program_id(0)
    nt = pl.program_id(1)

    x1 = xyz1_ref[0]
    x2 = xyz2_ref[0]

    x1r = x1.astype(jnp.bfloat16)
    x2r = x2.astype(jnp.bfloat16)
    x1sq = jnp.sum(x1 * x1, axis=0, keepdims=True)
    cross = lax.dot_general(x2r, x1r, (((0,), (0,)), ((), ())),
                            preferred_element_type=jnp.float32)
    x2sq_col = lax.dot_general(
        x2 * x2, jnp.ones((3, 1), jnp.float32), (((0,), (0,)), ((), ())),
        preferred_element_type=jnp.float32,
        precision=lax.Precision.HIGHEST)
    dT = (x1sq + x2sq_col) - 2.0 * cross

    iota = lax.broadcasted_iota(jnp.int32, (S, TN), 0)

    m1 = jnp.min(dT, axis=0, keepdims=True)
    i1 = jnp.min(jnp.where(dT == m1, iota, S), axis=0, keepdims=True)
    d2 = jnp.where(iota == i1, BIG, dT)
    m2 = jnp.min(d2, axis=0, keepdims=True)
    i2 = jnp.min(jnp.where(d2 == m2, iota, S), axis=0, keepdims=True)
    d3 = jnp.where(iota == i2, BIG, d2)
    m3 = jnp.min(d3, axis=0, keepdims=True)
    i3 = jnp.min(jnp.where(d3 == m3, iota, S), axis=0, keepdims=True)

    r1 = 1.0 / jnp.maximum(m1, 1e-8)
    r2 = 1.0 / jnp.maximum(m2, 1e-8)
    r3 = 1.0 / jnp.maximum(m3, 1e-8)
    norm = jnp.maximum(r1 + r2 + r3, 1e-8)

    wa_ref[0] = r1 / norm
    wb_ref[0] = r2 / norm
    wc_ref[0] = r3 / norm
    off = b * S
    ga_ref[0] = i1 + off
    gb_ref[0] = i2 + off
    gc_ref[0] = i3 + off

    @pl.when(nt == 0)
    def _():
        p2t_ref[...] = jnp.transpose(p2_ref[0], (1, 0))


def _sc_body(wa_hbm, wb_hbm, wc_hbm, ga_hbm, gb_hbm, gc_hbm, tbl_hbm,
             out_hbm, i0_v, i1_v, i2_v, w0_v, w1_v, w2_v, rows_v, out_v, sem):
    wid = lax.axis_index("s") * NC + lax.axis_index("c")
    qbase0 = wid * QPW

    pltpu.sync_copy(ga_hbm.at[pl.ds(qbase0, QPW)], i0_v)
    pltpu.sync_copy(gb_hbm.at[pl.ds(qbase0, QPW)], i1_v)
    pltpu.sync_copy(gc_hbm.at[pl.ds(qbase0, QPW)], i2_v)
    pltpu.sync_copy(wa_hbm.at[pl.ds(qbase0, QPW)], w0_v.at[pl.ds(0, QPW)])
    pltpu.sync_copy(wb_hbm.at[pl.ds(qbase0, QPW)], w1_v.at[pl.ds(0, QPW)])
    pltpu.sync_copy(wc_hbm.at[pl.ds(qbase0, QPW)], w2_v.at[pl.ds(0, QPW)])

    def chunk(c, carry):
        qbase = qbase0 + c * Q
        q0 = c * Q
        cp0 = pltpu.async_copy(tbl_hbm.at[i0_v.at[pl.ds(q0, Q)]],
                               rows_v.at[0], sem)
        cp1 = pltpu.async_copy(tbl_hbm.at[i1_v.at[pl.ds(q0, Q)]],
                               rows_v.at[1], sem)
        cp2 = pltpu.async_copy(tbl_hbm.at[i2_v.at[pl.ds(q0, Q)]],
                               rows_v.at[2], sem)
        cp0.wait()
        cp1.wait()
        cp2.wait()

        def per_q(q, carry2):
            w0 = w0_v[pl.ds(q0 + q, L)][0]
            w1 = w1_v[pl.ds(q0 + q, L)][0]
            w2 = w2_v[pl.ds(q0 + q, L)][0]
            for dd in range(D2 // L):
                sl = pl.ds(dd * L, L)
                acc = (w0 * rows_v[0, q, sl] + w1 * rows_v[1, q, sl]
                       + w2 * rows_v[2, q, sl])
                out_v[q, sl] = acc
            return carry2

        lax.fori_loop(0, Q, per_q, 0)
        pltpu.sync_copy(out_v, out_hbm.at[pl.ds(qbase, Q)])
        return carry

    lax.fori_loop(0, NCHUNK, chunk, 0)


def _k2_body(p1_ref, it_ref, w0_ref, z0_ref, s0_ref, q0_ref):
    b = pl.program_id(0)
    nt = pl.program_id(1)
    w0a = w0_ref[...][:, :D1]
    w0b = w0_ref[...][:, D1:]
    z0 = jnp.dot(w0a, p1_ref[0], preferred_element_type=jnp.float32)
    z0 = z0 + lax.dot_general(w0b, it_ref[...], (((1,), (1,)), ((), ())),
                              preferred_element_type=jnp.float32)
    z0_ref[0] = z0

    @pl.when(jnp.logical_and(b == 0, nt == 0))
    def _():
        s0_ref[...] = jnp.zeros_like(s0_ref)
        q0_ref[...] = jnp.zeros_like(q0_ref)

    s0_ref[...] += jnp.sum(z0, axis=1, keepdims=True)
    q0_ref[...] += jnp.sum(z0 * z0, axis=1, keepdims=True)


def _k3_body(z0_ref, s0_ref, q0_ref, g0_ref, b0_ref, w1_ref,
             z1_ref, s1_ref, q1_ref):
    b = pl.program_id(0)
    nt = pl.program_id(1)
    mean = s0_ref[...] * (1.0 / CNT)
    var = q0_ref[...] * (1.0 / CNT) - mean * mean
    a = g0_ref[...] * lax.rsqrt(var + EPS)
    c = b0_ref[...] - mean * a
    h = jnp.maximum(z0_ref[0] * a + c, 0.0)
    z1 = jnp.dot(w1_ref[...], h, preferred_element_type=jnp.float32)
    z1_ref[0] = z1

    @pl.when(jnp.logical_and(b == 0, nt == 0))
    def _():
        s1_ref[...] = jnp.zeros_like(s1_ref)
        q1_ref[...] = jnp.zeros_like(q1_ref)

    s1_ref[...] += jnp.sum(z1, axis=1, keepdims=True)
    q1_ref[...] += jnp.sum(z1 * z1, axis=1, keepdims=True)


def _k4_body(z1_ref, s1_ref, q1_ref, g1_ref, b1_ref, out_ref):
    mean = s1_ref[...] * (1.0 / CNT)
    var = q1_ref[...] * (1.0 / CNT) - mean * mean
    a = g1_ref[...] * lax.rsqrt(var + EPS)
    c = b1_ref[...] - mean * a
    out_ref[0] = jnp.maximum(z1_ref[0] * a + c, 0.0)


@jax.jit
def kernel(xyz1, xyz2, points1, points2, W0, g0, b0, W1, g1, b1):
    g0c = g0.reshape(H0, 1)
    b0c = b0.reshape(H0, 1)
    g1c = g1.reshape(H1, 1)
    b1c = b1.reshape(H1, 1)

    wa, wb, wc, ga, gb, gc, p2t = pl.pallas_call(
        _k1_body,
        grid=(B, NT),
        in_specs=[
            pl.BlockSpec((1, 3, TN), lambda b, n: (b, 0, n)),
            pl.BlockSpec((1, 3, S), lambda b, n: (b, 0, 0)),
            pl.BlockSpec((1, D2, S), lambda b, n: (b, 0, 0)),
        ],
        out_specs=[
            pl.BlockSpec((1, 1, TN), lambda b, n: (b, 0, n)),
            pl.BlockSpec((1, 1, TN), lambda b, n: (b, 0, n)),
            pl.BlockSpec((1, 1, TN), lambda b, n: (b, 0, n)),
            pl.BlockSpec((1, 1, TN), lambda b, n: (b, 0, n)),
            pl.BlockSpec((1, 1, TN), lambda b, n: (b, 0, n)),
            pl.BlockSpec((1, 1, TN), lambda b, n: (b, 0, n)),
            pl.BlockSpec((S, D2), lambda b, n: (b, 0)),
        ],
        out_shape=[
            jax.ShapeDtypeStruct((B, 1, N), jnp.float32),
            jax.ShapeDtypeStruct((B, 1, N), jnp.float32),
            jax.ShapeDtypeStruct((B, 1, N), jnp.float32),
            jax.ShapeDtypeStruct((B, 1, N), jnp.int32),
            jax.ShapeDtypeStruct((B, 1, N), jnp.int32),
            jax.ShapeDtypeStruct((B, 1, N), jnp.int32),
            jax.ShapeDtypeStruct((B * S, D2), jnp.float32),
        ],
    )(xyz1, xyz2, points2)

    mesh = plsc.VectorSubcoreMesh(core_axis_name="c", subcore_axis_name="s")
    waf = wa.reshape(B * N)
    wbf = wb.reshape(B * N)
    wcf = wc.reshape(B * N)
    gaf = ga.reshape(B * N)
    gbf = gb.reshape(B * N)
    gcf = gc.reshape(B * N)
    interp = pl.kernel(
        _sc_body,
        mesh=mesh,
        out_type=jax.ShapeDtypeStruct((B * N, D2), jnp.float32),
        scratch_types=[
            pltpu.VMEM((QPW,), jnp.int32),
            pltpu.VMEM((QPW,), jnp.int32),
            pltpu.VMEM((QPW,), jnp.int32),
            pltpu.VMEM((QPW + L,), jnp.float32),
            pltpu.VMEM((QPW + L,), jnp.float32),
            pltpu.VMEM((QPW + L,), jnp.float32),
            pltpu.VMEM((3, Q, D2), jnp.float32),
            pltpu.VMEM((Q, D2), jnp.float32),
            pltpu.SemaphoreType.DMA,
        ],
    )(waf, wbf, wcf, gaf, gbf, gcf, p2t)

    z0, s0, q0 = pl.pallas_call(
        _k2_body,
        grid=(B, NT),
        in_specs=[
            pl.BlockSpec((1, D1, TN), lambda b, n: (b, 0, n)),
            pl.BlockSpec((TN, D2), lambda b, n: (b * NT + n, 0)),
            pl.BlockSpec((H0, D1 + D2), lambda b, n: (0, 0)),
        ],
        out_specs=[
            pl.BlockSpec((1, H0, TN), lambda b, n: (b, 0, n)),
            pl.BlockSpec((H0, 1), lambda b, n: (0, 0)),
            pl.BlockSpec((H0, 1), lambda b, n: (0, 0)),
        ],
        out_shape=[
            jax.ShapeDtypeStruct((B, H0, N), jnp.float32),
            jax.ShapeDtypeStruct((H0, 1), jnp.float32),
            jax.ShapeDtypeStruct((H0, 1), jnp.float32),
        ],
    )(points1, interp, W0)

    z1, s1, q1 = pl.pallas_call(
        _k3_body,
        grid=(B, NT),
        in_specs=[
            pl.BlockSpec((1, H0, TN), lambda b, n: (b, 0, n)),
            pl.BlockSpec((H0, 1), lambda b, n: (0, 0)),
            pl.BlockSpec((H0, 1), lambda b, n: (0, 0)),
            pl.BlockSpec((H0, 1), lambda b, n: (0, 0)),
            pl.BlockSpec((H0, 1), lambda b, n: (0, 0)),
            pl.BlockSpec((H1, H0), lambda b, n: (0, 0)),
        ],
        out_specs=[
            pl.BlockSpec((1, H1, TN), lambda b, n: (b, 0, n)),
            pl.BlockSpec((H1, 1), lambda b, n: (0, 0)),
            pl.BlockSpec((H1, 1), lambda b, n: (0, 0)),
        ],
        out_shape=[
            jax.ShapeDtypeStruct((B, H1, N), jnp.float32),
            jax.ShapeDtypeStruct((H1, 1), jnp.float32),
            jax.ShapeDtypeStruct((H1, 1), jnp.float32),
        ],
    )(z0, s0, q0, g0c, b0c, W1)

    out = pl.pallas_call(
        _k4_body,
        grid=(B, NT),
        in_specs=[
            pl.BlockSpec((1, H1, TN), lambda b, n: (b, 0, n)),
            pl.BlockSpec((H1, 1), lambda b, n: (0, 0)),
            pl.BlockSpec((H1, 1), lambda b, n: (0, 0)),
            pl.BlockSpec((H1, 1), lambda b, n: (0, 0)),
            pl.BlockSpec((H1, 1), lambda b, n: (0, 0)),
        ],
        out_specs=pl.BlockSpec((1, H1, TN), lambda b, n: (b, 0, n)),
        out_shape=jax.ShapeDtypeStruct((B, H1, N), jnp.float32),
    )(z1, s1, q1, g1c, b1c)

    return out


# speedup vs baseline: 10.8483x; 1.0188x over previous
import functools

import jax
import jax.numpy as jnp
from jax import lax
from jax.experimental import pallas as pl
from jax.experimental.pallas import tpu as pltpu, tpu_sc as plsc


B, N, S, D1, D2 = 8, 4096, 1024, 256, 256
B2 = B // 2
H0, H1 = 512, 256
TN = 256
NT = N // TN
CNT = float(B * N)
EPS = 1e-5
BIG = 3.0e38

NC, NS, L = 2, 16, 16
NW = NC * NS
QPW = (B2 * N) // NW
Q = 64
NCHUNK = QPW // Q


def _k1_body(xyz1_ref, xyz2_ref, p2_ref,
             wa_ref, wb_ref, wc_ref, ga_ref, gb_ref, gc_ref, p2t_ref):
    b = pl.program_id(0)
    nt = pl.program_id(1)

    x1 = xyz1_ref[0]
    x2 = xyz2_ref[0]

    x1r = x1.astype(jnp.bfloat16)
    x2r = x2.astype(jnp.bfloat16)
    x1sq = jnp.sum(x1 * x1, axis=0, keepdims=True)
    cross = lax.dot_general(x2r, x1r, (((0,), (0,)), ((), ())),
                            preferred_element_type=jnp.float32)
    x2sq_col = lax.dot_general(
        x2 * x2, jnp.ones((3, 1), jnp.float32), (((0,), (0,)), ((), ())),
        preferred_element_type=jnp.float32,
        precision=lax.Precision.HIGHEST)
    dT = (x1sq + x2sq_col) - 2.0 * cross

    iota = lax.broadcasted_iota(jnp.int32, (S, TN), 0)

    m1 = jnp.min(dT, axis=0, keepdims=True)
    i1 = jnp.min(jnp.where(dT == m1, iota, S), axis=0, keepdims=True)
    d2 = jnp.where(iota == i1, BIG, dT)
    m2 = jnp.min(d2, axis=0, keepdims=True)
    i2 = jnp.min(jnp.where(d2 == m2, iota, S), axis=0, keepdims=True)
    d3 = jnp.where(iota == i2, BIG, d2)
    m3 = jnp.min(d3, axis=0, keepdims=True)
    i3 = jnp.min(jnp.where(d3 == m3, iota, S), axis=0, keepdims=True)

    r1 = 1.0 / jnp.maximum(m1, 1e-8)
    r2 = 1.0 / jnp.maximum(m2, 1e-8)
    r3 = 1.0 / jnp.maximum(m3, 1e-8)
    norm = jnp.maximum(r1 + r2 + r3, 1e-8)

    wa_ref[0] = r1 / norm
    wb_ref[0] = r2 / norm
    wc_ref[0] = r3 / norm
    off = b * S
    ga_ref[0] = i1 + off
    gb_ref[0] = i2 + off
    gc_ref[0] = i3 + off

    @pl.when(nt == 0)
    def _():
        p2t_ref[...] = jnp.transpose(p2_ref[0], (1, 0))


def _sc_body(wa_hbm, wb_hbm, wc_hbm, ga_hbm, gb_hbm, gc_hbm, tbl_hbm,
             out_hbm, i0_v, i1_v, i2_v, w0_v, w1_v, w2_v, rows_v, out_v, sem):
    wid = lax.axis_index("s") * NC + lax.axis_index("c")
    qbase0 = wid * QPW

    pltpu.sync_copy(ga_hbm.at[pl.ds(qbase0, QPW)], i0_v)
    pltpu.sync_copy(gb_hbm.at[pl.ds(qbase0, QPW)], i1_v)
    pltpu.sync_copy(gc_hbm.at[pl.ds(qbase0, QPW)], i2_v)
    pltpu.sync_copy(wa_hbm.at[pl.ds(qbase0, QPW)], w0_v.at[pl.ds(0, QPW)])
    pltpu.sync_copy(wb_hbm.at[pl.ds(qbase0, QPW)], w1_v.at[pl.ds(0, QPW)])
    pltpu.sync_copy(wc_hbm.at[pl.ds(qbase0, QPW)], w2_v.at[pl.ds(0, QPW)])

    def chunk(c, carry):
        qbase = qbase0 + c * Q
        q0 = c * Q
        cp0 = pltpu.async_copy(tbl_hbm.at[i0_v.at[pl.ds(q0, Q)]],
                               rows_v.at[0], sem)
        cp1 = pltpu.async_copy(tbl_hbm.at[i1_v.at[pl.ds(q0, Q)]],
                               rows_v.at[1], sem)
        cp2 = pltpu.async_copy(tbl_hbm.at[i2_v.at[pl.ds(q0, Q)]],
                               rows_v.at[2], sem)
        cp0.wait()
        cp1.wait()
        cp2.wait()

        def per_q(q, carry2):
            w0 = w0_v[pl.ds(q0 + q, L)][0]
            w1 = w1_v[pl.ds(q0 + q, L)][0]
            w2 = w2_v[pl.ds(q0 + q, L)][0]
            for dd in range(D2 // L):
                sl = pl.ds(dd * L, L)
                acc = (w0 * rows_v[0, q, sl] + w1 * rows_v[1, q, sl]
                       + w2 * rows_v[2, q, sl])
                out_v[q, sl] = acc
            return carry2

        lax.fori_loop(0, Q, per_q, 0)
        pltpu.sync_copy(out_v, out_hbm.at[pl.ds(qbase, Q)])
        return carry

    lax.fori_loop(0, NCHUNK, chunk, 0)


def _k2_body(p1_ref, it_ref, w0_ref, z0_ref, s0_ref, q0_ref):
    b = pl.program_id(0)
    nt = pl.program_id(1)
    w0a = w0_ref[...][:, :D1]
    w0b = w0_ref[...][:, D1:]
    z0 = jnp.dot(w0a, p1_ref[0], preferred_element_type=jnp.float32)
    z0 = z0 + lax.dot_general(w0b, it_ref[...], (((1,), (1,)), ((), ())),
                              preferred_element_type=jnp.float32)
    z0_ref[0] = z0

    @pl.when(jnp.logical_and(b == 0, nt == 0))
    def _():
        s0_ref[...] = jnp.zeros_like(s0_ref)
        q0_ref[...] = jnp.zeros_like(q0_ref)

    s0_ref[...] += jnp.sum(z0, axis=1, keepdims=True)
    q0_ref[...] += jnp.sum(z0 * z0, axis=1, keepdims=True)


def _k3_body(z0_ref, s0a_ref, q0a_ref, s0b_ref, q0b_ref, g0_ref, b0_ref,
             w1_ref, z1_ref, s1_ref, q1_ref):
    b = pl.program_id(0)
    nt = pl.program_id(1)
    mean = (s0a_ref[...] + s0b_ref[...]) * (1.0 / CNT)
    var = (q0a_ref[...] + q0b_ref[...]) * (1.0 / CNT) - mean * mean
    a = g0_ref[...] * lax.rsqrt(var + EPS)
    c = b0_ref[...] - mean * a
    h = jnp.maximum(z0_ref[0] * a + c, 0.0)
    z1 = jnp.dot(w1_ref[...], h, preferred_element_type=jnp.float32)
    z1_ref[0] = z1

    @pl.when(jnp.logical_and(b == 0, nt == 0))
    def _():
        s1_ref[...] = jnp.zeros_like(s1_ref)
        q1_ref[...] = jnp.zeros_like(q1_ref)

    s1_ref[...] += jnp.sum(z1, axis=1, keepdims=True)
    q1_ref[...] += jnp.sum(z1 * z1, axis=1, keepdims=True)


def _k4_body(z1_ref, s1a_ref, q1a_ref, s1b_ref, q1b_ref, g1_ref, b1_ref,
             out_ref):
    mean = (s1a_ref[...] + s1b_ref[...]) * (1.0 / CNT)
    var = (q1a_ref[...] + q1b_ref[...]) * (1.0 / CNT) - mean * mean
    a = g1_ref[...] * lax.rsqrt(var + EPS)
    c = b1_ref[...] - mean * a
    out_ref[0] = jnp.maximum(z1_ref[0] * a + c, 0.0)


def _front_half(xyz1h, xyz2h, p2h):
    wa, wb, wc, ga, gb, gc, p2t = pl.pallas_call(
        _k1_body,
        grid=(B2, NT),
        in_specs=[
            pl.BlockSpec((1, 3, TN), lambda b, n: (b, 0, n)),
            pl.BlockSpec((1, 3, S), lambda b, n: (b, 0, 0)),
            pl.BlockSpec((1, D2, S), lambda b, n: (b, 0, 0)),
        ],
        out_specs=[
            pl.BlockSpec((1, 1, TN), lambda b, n: (b, 0, n)),
            pl.BlockSpec((1, 1, TN), lambda b, n: (b, 0, n)),
            pl.BlockSpec((1, 1, TN), lambda b, n: (b, 0, n)),
            pl.BlockSpec((1, 1, TN), lambda b, n: (b, 0, n)),
            pl.BlockSpec((1, 1, TN), lambda b, n: (b, 0, n)),
            pl.BlockSpec((1, 1, TN), lambda b, n: (b, 0, n)),
            pl.BlockSpec((S, D2), lambda b, n: (b, 0)),
        ],
        out_shape=[
            jax.ShapeDtypeStruct((B2, 1, N), jnp.float32),
            jax.ShapeDtypeStruct((B2, 1, N), jnp.float32),
            jax.ShapeDtypeStruct((B2, 1, N), jnp.float32),
            jax.ShapeDtypeStruct((B2, 1, N), jnp.int32),
            jax.ShapeDtypeStruct((B2, 1, N), jnp.int32),
            jax.ShapeDtypeStruct((B2, 1, N), jnp.int32),
            jax.ShapeDtypeStruct((B2 * S, D2), jnp.float32),
        ],
    )(xyz1h, xyz2h, p2h)

    mesh = plsc.VectorSubcoreMesh(core_axis_name="c", subcore_axis_name="s")
    interp = pl.kernel(
        _sc_body,
        mesh=mesh,
        out_type=jax.ShapeDtypeStruct((B2 * N, D2), jnp.float32),
        scratch_types=[
            pltpu.VMEM((QPW,), jnp.int32),
            pltpu.VMEM((QPW,), jnp.int32),
            pltpu.VMEM((QPW,), jnp.int32),
            pltpu.VMEM((QPW + L,), jnp.float32),
            pltpu.VMEM((QPW + L,), jnp.float32),
            pltpu.VMEM((QPW + L,), jnp.float32),
            pltpu.VMEM((3, Q, D2), jnp.float32),
            pltpu.VMEM((Q, D2), jnp.float32),
            pltpu.SemaphoreType.DMA,
        ],
    )(wa.reshape(B2 * N), wb.reshape(B2 * N), wc.reshape(B2 * N),
      ga.reshape(B2 * N), gb.reshape(B2 * N), gc.reshape(B2 * N), p2t)
    return interp


def _layer0_half(p1h, interp, W0):
    return pl.pallas_call(
        _k2_body,
        grid=(B2, NT),
        in_specs=[
            pl.BlockSpec((1, D1, TN), lambda b, n: (b, 0, n)),
            pl.BlockSpec((TN, D2), lambda b, n: (b * NT + n, 0)),
            pl.BlockSpec((H0, D1 + D2), lambda b, n: (0, 0)),
        ],
        out_specs=[
            pl.BlockSpec((1, H0, TN), lambda b, n: (b, 0, n)),
            pl.BlockSpec((H0, 1), lambda b, n: (0, 0)),
            pl.BlockSpec((H0, 1), lambda b, n: (0, 0)),
        ],
        out_shape=[
            jax.ShapeDtypeStruct((B2, H0, N), jnp.float32),
            jax.ShapeDtypeStruct((H0, 1), jnp.float32),
            jax.ShapeDtypeStruct((H0, 1), jnp.float32),
        ],
    )(p1h, interp, W0)


def _layer1_half(z0h, s0a, q0a, s0b, q0b, g0c, b0c, W1):
    return pl.pallas_call(
        _k3_body,
        grid=(B2, NT),
        in_specs=[
            pl.BlockSpec((1, H0, TN), lambda b, n: (b, 0, n)),
            pl.BlockSpec((H0, 1), lambda b, n: (0, 0)),
            pl.BlockSpec((H0, 1), lambda b, n: (0, 0)),
            pl.BlockSpec((H0, 1), lambda b, n: (0, 0)),
            pl.BlockSpec((H0, 1), lambda b, n: (0, 0)),
            pl.BlockSpec((H0, 1), lambda b, n: (0, 0)),
            pl.BlockSpec((H0, 1), lambda b, n: (0, 0)),
            pl.BlockSpec((H1, H0), lambda b, n: (0, 0)),
        ],
        out_specs=[
            pl.BlockSpec((1, H1, TN), lambda b, n: (b, 0, n)),
            pl.BlockSpec((H1, 1), lambda b, n: (0, 0)),
            pl.BlockSpec((H1, 1), lambda b, n: (0, 0)),
        ],
        out_shape=[
            jax.ShapeDtypeStruct((B2, H1, N), jnp.float32),
            jax.ShapeDtypeStruct((H1, 1), jnp.float32),
            jax.ShapeDtypeStruct((H1, 1), jnp.float32),
        ],
    )(z0h, s0a, q0a, s0b, q0b, g0c, b0c, W1)


def _out_half(z1h, s1a, q1a, s1b, q1b, g1c, b1c):
    return pl.pallas_call(
        _k4_body,
        grid=(B2, NT),
        in_specs=[
            pl.BlockSpec((1, H1, TN), lambda b, n: (b, 0, n)),
            pl.BlockSpec((H1, 1), lambda b, n: (0, 0)),
            pl.BlockSpec((H1, 1), lambda b, n: (0, 0)),
            pl.BlockSpec((H1, 1), lambda b, n: (0, 0)),
            pl.BlockSpec((H1, 1), lambda b, n: (0, 0)),
            pl.BlockSpec((H1, 1), lambda b, n: (0, 0)),
            pl.BlockSpec((H1, 1), lambda b, n: (0, 0)),
        ],
        out_specs=pl.BlockSpec((1, H1, TN), lambda b, n: (b, 0, n)),
        out_shape=jax.ShapeDtypeStruct((B2, H1, N), jnp.float32),
    )(z1h, s1a, q1a, s1b, q1b, g1c, b1c)


@jax.jit
def kernel(xyz1, xyz2, points1, points2, W0, g0, b0, W1, g1, b1):
    g0c = g0.reshape(H0, 1)
    b0c = b0.reshape(H0, 1)
    g1c = g1.reshape(H1, 1)
    b1c = b1.reshape(H1, 1)

    interp_a = _front_half(xyz1[:B2], xyz2[:B2], points2[:B2])
    interp_b = _front_half(xyz1[B2:], xyz2[B2:], points2[B2:])

    z0a, s0a, q0a = _layer0_half(points1[:B2], interp_a, W0)
    z0b, s0b, q0b = _layer0_half(points1[B2:], interp_b, W0)

    z1a, s1a, q1a = _layer1_half(z0a, s0a, q0a, s0b, q0b, g0c, b0c, W1)
    z1b, s1b, q1b = _layer1_half(z0b, s0a, q0a, s0b, q0b, g0c, b0c, W1)

    outa = _out_half(z1a, s1a, q1a, s1b, q1b, g1c, b1c)
    outb = _out_half(z1b, s1a, q1a, s1b, q1b, g1c, b1c)

    return jnp.concatenate([outa, outb], axis=0)
